# Initial kernel scaffold; baseline (speedup 1.0000x reference)
import jax
import jax.numpy as jnp
from jax.experimental import pallas as pl


def kernel(X, C, B):
    raise NotImplementedError("write your pallas kernel here")


# speedup vs baseline: 3.8238x; 3.8238x over previous
import functools

import jax
import jax.numpy as jnp
from jax import lax
from jax.experimental import pallas as pl
from jax.experimental.pallas import tpu as pltpu, tpu_sc as plsc

N, D, M, V = 32768, 256, 8, 8192
NC, NS, L = 2, 16, 16
NW = NC * NS
ROWS_PER_W = N // NW
R = 16
BLOCKS = ROWS_PER_W // R
IDXB = R * M


def _sc_body(x_hbm, c_hbm, b_hbm, out_hbm,
             idx_v, rows_v, x_v, err_v, out_v, gsem, xsem):
    wid = lax.axis_index("c") * NS + lax.axis_index("s")
    row0 = wid * ROWS_PER_W

    def _block(blk, carry):
        base = row0 + blk * R
        pltpu.sync_copy(b_hbm.at[pl.ds(base * M, IDXB)], idx_v)
        gcopy = pltpu.async_copy(c_hbm.at[idx_v], rows_v, gsem)
        xcopy = pltpu.async_copy(x_hbm.at[pl.ds(base, R)], x_v, xsem)
        gcopy.wait()
        xcopy.wait()

        lane = lax.iota(jnp.int32, L)

        def _row(r, acc):
            g = r * M
            err = jnp.zeros((L,), jnp.float32)
            for k in range(D // L):
                col = pl.ds(k * L, L)
                s = rows_v[g, col]
                for j in range(1, M):
                    s = s + rows_v[g + j, col]
                dlt = x_v[r, col] - s
                err = err + dlt * dlt
            return jnp.where(lane == r, jnp.sum(err), acc)

        acc = lax.fori_loop(0, R, _row, jnp.zeros((L,), jnp.float32))
        out_v[pl.ds(blk * R, R)] = acc
        return carry

    lax.fori_loop(0, BLOCKS, _block, 0)
    pltpu.sync_copy(out_v, out_hbm.at[pl.ds(row0, ROWS_PER_W)])


@functools.lru_cache(maxsize=1)
def _build():
    return pl.kernel(
        _sc_body,
        out_type=jax.ShapeDtypeStruct((N,), jnp.float32),
        mesh=plsc.VectorSubcoreMesh(core_axis_name="c", subcore_axis_name="s",
                                    num_cores=NC, num_subcores=NS),
        compiler_params=pltpu.CompilerParams(needs_layout_passes=False),
        scratch_types=[
            pltpu.VMEM((IDXB,), jnp.int32),
            pltpu.VMEM((IDXB, D), jnp.float32),
            pltpu.VMEM((R, D), jnp.float32),
            pltpu.VMEM((R * L,), jnp.float32),
            pltpu.VMEM((ROWS_PER_W,), jnp.float32),
            pltpu.SemaphoreType.DMA,
            pltpu.SemaphoreType.DMA,
        ],
    )


def kernel(X, C, B):
    return _build()(X, C, B.reshape(-1))


# speedup vs baseline: 6.8444x; 1.7900x over previous
import functools

import jax
import jax.numpy as jnp
from jax import lax
from jax.experimental import pallas as pl
from jax.experimental.pallas import tpu as pltpu, tpu_sc as plsc

N, D, M, V = 32768, 256, 8, 8192
NC, NS, L = 2, 16, 16
NW = NC * NS
ROWS_PER_W = N // NW
R = 16
BLOCKS = ROWS_PER_W // R
IDXB = R * M


def _sc_body(x_hbm, c_hbm, b_hbm, out_hbm,
             idx_all, rows0, rows1, x0, x1, out_v, gs0, gs1, xs0, xs1):
    wid = lax.axis_index("c") * NS + lax.axis_index("s")
    row0 = wid * ROWS_PER_W

    pltpu.sync_copy(b_hbm.at[pl.ds(row0 * M, ROWS_PER_W * M)], idx_all)

    rows = (rows0, rows1)
    xbuf = (x0, x1)
    gsem = (gs0, gs1)
    xsem = (xs0, xs1)

    def idx_slice(blk):
        return idx_all.at[pl.ds(pl.multiple_of(blk * IDXB, IDXB), IDXB)]

    def start(blk, buf):
        base = row0 + blk * R
        pltpu.async_copy(c_hbm.at[idx_slice(blk)], rows[buf], gsem[buf])
        pltpu.async_copy(x_hbm.at[pl.ds(base, R)], xbuf[buf], xsem[buf])

    def wait(blk, buf):
        pltpu.make_async_copy(
            c_hbm.at[idx_slice(blk)], rows[buf], gsem[buf]).wait()
        pltpu.make_async_copy(
            x_hbm.at[pl.ds(row0, R)], xbuf[buf], xsem[buf]).wait()

    lane = lax.iota(jnp.int32, L)

    def compute(blk, buf):
        rows_v = rows[buf]
        x_v = xbuf[buf]

        def _row(r, acc):
            g = r * M
            err = jnp.zeros((L,), jnp.float32)
            for k in range(D // L):
                col = pl.ds(k * L, L)
                s = rows_v[g, col]
                for j in range(1, M):
                    s = s + rows_v[g + j, col]
                dlt = x_v[r, col] - s
                err = err + dlt * dlt
            return jnp.where(lane == r, jnp.sum(err), acc)

        acc = lax.fori_loop(0, R, _row, jnp.zeros((L,), jnp.float32))
        out_v[pl.ds(blk * R, R)] = acc

    start(0, 0)
    start(1, 1)

    def _pair(p, carry):
        b0 = 2 * p
        wait(b0, 0)
        compute(b0, 0)

        @pl.when(b0 + 2 < BLOCKS)
        def _():
            start(b0 + 2, 0)

        wait(b0 + 1, 1)
        compute(b0 + 1, 1)

        @pl.when(b0 + 3 < BLOCKS)
        def _():
            start(b0 + 3, 1)

        return carry

    lax.fori_loop(0, BLOCKS // 2, _pair, 0)
    pltpu.sync_copy(out_v, out_hbm.at[pl.ds(row0, ROWS_PER_W)])


@functools.lru_cache(maxsize=1)
def _build():
    return pl.kernel(
        _sc_body,
        out_type=jax.ShapeDtypeStruct((N,), jnp.float32),
        mesh=plsc.VectorSubcoreMesh(core_axis_name="c", subcore_axis_name="s",
                                    num_cores=NC, num_subcores=NS),
        compiler_params=pltpu.CompilerParams(needs_layout_passes=False),
        scratch_types=[
            pltpu.VMEM((ROWS_PER_W * M,), jnp.int32),
            pltpu.VMEM((IDXB, D), jnp.float32),
            pltpu.VMEM((IDXB, D), jnp.float32),
            pltpu.VMEM((R, D), jnp.float32),
            pltpu.VMEM((R, D), jnp.float32),
            pltpu.VMEM((ROWS_PER_W,), jnp.float32),
            pltpu.SemaphoreType.DMA,
            pltpu.SemaphoreType.DMA,
            pltpu.SemaphoreType.DMA,
            pltpu.SemaphoreType.DMA,
        ],
    )


def kernel(X, C, B):
    return _build()(X, C, B.reshape(-1))
